# Initial kernel scaffold; baseline (speedup 1.0000x reference)
import jax
import jax.numpy as jnp
from jax.experimental import pallas as pl


def kernel(x, edge_index, W1, b1, W2, b2, lin_W, lin_b):
    raise NotImplementedError("write your pallas kernel here")


# speedup vs baseline: 49.4234x; 49.4234x over previous
import functools

import jax
import jax.numpy as jnp
from jax import lax
from jax.experimental import pallas as pl
from jax.experimental.pallas import tpu as pltpu
from jax.experimental.pallas import tpu_sc as plsc

N = 10000
E = 320000
F_IN = 128
H = 16
OUT = 10

NC = 2
NS = 16
NW = NC * NS
NPAD = 10240
SL = NPAD // NS
CHUNK = 128
G = 8
GS = 8
EP = 327680
RW = EP // (NW * CHUNK)

_mesh = plsc.VectorSubcoreMesh(core_axis_name="c", subcore_axis_name="s")


@functools.partial(
    pl.kernel,
    mesh=_mesh,
    out_type=jax.ShapeDtypeStruct((NC, NPAD), jnp.float32),
    scratch_types=[
        pltpu.VMEM((G, CHUNK), jnp.int32),
        pltpu.VMEM((CHUNK,), jnp.float32),
        pltpu.VMEM_SHARED((NPAD,), jnp.float32),
    ],
)
def _deg_kernel(col_hbm, zeros_hbm, out_hbm, idx_v, ones_v, deg_sh):
    c = lax.axis_index("c")
    s = lax.axis_index("s")
    w = c * NS + s
    for k in range(CHUNK // 16):
        ones_v[pl.ds(k * 16, 16)] = jnp.ones((16,), jnp.float32)
    pltpu.sync_copy(zeros_hbm.at[pl.ds(s * SL, SL)], deg_sh.at[pl.ds(s * SL, SL)])
    plsc.subcore_barrier()

    def outer(i, carry):
        base = w * RW + i * G
        pltpu.sync_copy(col_hbm.at[pl.ds(base, G)], idx_v)
        for j in range(G):
            pltpu.sync_copy(ones_v, deg_sh.at[idx_v.at[j]], add=True)
        return carry

    lax.fori_loop(0, RW // G, outer, 0)
    plsc.subcore_barrier()
    pltpu.sync_copy(deg_sh.at[pl.ds(s * SL, SL)], out_hbm.at[c, pl.ds(s * SL, SL)])


@functools.partial(
    pl.kernel,
    mesh=_mesh,
    out_type=(
        jax.ShapeDtypeStruct((NC, NPAD, H), jnp.float32),
        jax.ShapeDtypeStruct((NC, NPAD), jnp.float32),
    ),
    scratch_types=[
        pltpu.VMEM((GS, CHUNK), jnp.int32),
        pltpu.VMEM((GS, CHUNK), jnp.int32),
        pltpu.VMEM((CHUNK, H), jnp.float32),
        pltpu.VMEM((CHUNK,), jnp.float32),
        pltpu.VMEM_SHARED((NPAD, H), jnp.float32),
        pltpu.VMEM_SHARED((NPAD,), jnp.float32),
        pltpu.SemaphoreType.DMA,
        pltpu.SemaphoreType.DMA,
    ],
    compiler_params=pltpu.CompilerParams(use_tc_tiling_on_sc=False),
)
def _scatter_kernel(row_hbm, col_hbm, y_hbm, dinv_hbm, z1_hbm, z2_hbm,
                    acc_out, t_out,
                    ridx_v, cidx_v, rows_v, dv_v, acc_sh, t_sh, sem, sem2):
    c = lax.axis_index("c")
    s = lax.axis_index("s")
    w = c * NS + s
    pltpu.sync_copy(z2_hbm.at[pl.ds(s * SL, SL)], acc_sh.at[pl.ds(s * SL, SL)])
    pltpu.sync_copy(z1_hbm.at[pl.ds(s * SL, SL)], t_sh.at[pl.ds(s * SL, SL)])
    plsc.subcore_barrier()

    def inner(j, carry):
        cp1 = pltpu.async_copy(y_hbm.at[ridx_v.at[j]], rows_v, sem)
        cp2 = pltpu.async_copy(dinv_hbm.at[cidx_v.at[j]], dv_v, sem2)
        cp1.wait()
        pltpu.sync_copy(rows_v, acc_sh.at[cidx_v.at[j]], add=True)
        cp2.wait()
        pltpu.sync_copy(dv_v, t_sh.at[ridx_v.at[j]], add=True)
        return carry

    def outer(i, carry):
        base = w * RW + i * GS
        pltpu.sync_copy(row_hbm.at[pl.ds(base, GS)], ridx_v)
        pltpu.sync_copy(col_hbm.at[pl.ds(base, GS)], cidx_v)
        return lax.fori_loop(0, GS, inner, carry)

    lax.fori_loop(0, RW // GS, outer, 0)
    plsc.subcore_barrier()
    pltpu.sync_copy(acc_sh.at[pl.ds(s * SL, SL)], acc_out.at[c, pl.ds(s * SL, SL)])
    pltpu.sync_copy(t_sh.at[pl.ds(s * SL, SL)], t_out.at[c, pl.ds(s * SL, SL)])


def _scale_body(x_ref, w1_ref, degp_ref, y_ref, dinv_ref):
    deg = degp_ref[:, 0:1] + degp_ref[:, 1:2] + 1.0
    dinv = lax.rsqrt(deg)
    dinv_ref[...] = dinv
    xw = jnp.dot(x_ref[...], w1_ref[...], preferred_element_type=jnp.float32)
    y_ref[...] = xw * dinv


_scale_call = pl.pallas_call(
    _scale_body,
    out_shape=(
        jax.ShapeDtypeStruct((NPAD, H), jnp.float32),
        jax.ShapeDtypeStruct((NPAD, 1), jnp.float32),
    ),
)


def _final_body(accp_ref, tp_ref, y_ref, dinv_ref, b1_ref, w2_ref, b2_ref,
                lw_ref, lb_ref, out_ref):
    dinv = dinv_ref[...]
    acc = accp_ref[0] + accp_ref[1]
    h = jnp.maximum(dinv * (acc + y_ref[...]) + b1_ref[...], 0.0)
    t = tp_ref[:, 0:1] + tp_ref[:, 1:2]
    s = dinv * (dinv + t)
    ridx = lax.broadcasted_iota(jnp.int32, (NPAD, 1), 0)
    s = jnp.where(ridx < N, s, 0.0)
    z = jnp.sum(s * h, axis=0, keepdims=True)
    pooled = jnp.dot(z * (1.0 / N), w2_ref[...],
                     preferred_element_type=jnp.float32) + b2_ref[...]
    out_ref[...] = jnp.dot(pooled, lw_ref[...],
                           preferred_element_type=jnp.float32) + lb_ref[...]


_final_call = pl.pallas_call(
    _final_body,
    out_shape=jax.ShapeDtypeStruct((1, OUT), jnp.float32),
)


def kernel(x, edge_index, W1, b1, W2, b2, lin_W, lin_b):
    row = edge_index[0]
    col = edge_index[1]
    npad_extra = NPAD - N
    pad_idx = (N + jnp.arange(EP - E, dtype=jnp.int32) % npad_extra)
    rowp = jnp.concatenate([row, pad_idx]).reshape(EP // CHUNK, CHUNK)
    colp = jnp.concatenate([col, pad_idx]).reshape(EP // CHUNK, CHUNK)
    zeros1 = jnp.zeros((NPAD,), jnp.float32)
    zeros2 = jnp.zeros((NPAD, H), jnp.float32)

    degp = _deg_kernel(colp, zeros1)
    xp = jnp.pad(x, ((0, npad_extra), (0, 0)))
    y, dinv2 = _scale_call(xp, W1, degp.T)
    dinv1 = dinv2.reshape(NPAD)
    accp, tp = _scatter_kernel(rowp, colp, y, dinv1, zeros1, zeros2)
    out = _final_call(accp, tp.T, y, dinv2, b1.reshape(1, H), W2,
                      b2.reshape(1, H), lin_W, lin_b.reshape(1, OUT))
    return out.reshape(OUT)


# speedup vs baseline: 58.6319x; 1.1863x over previous
import functools

import jax
import jax.numpy as jnp
from jax import lax
from jax.experimental import pallas as pl
from jax.experimental.pallas import tpu as pltpu
from jax.experimental.pallas import tpu_sc as plsc

N = 10000
E = 320000
F_IN = 128
H = 16
OUT = 10

NC = 2
NS = 16
NW = NC * NS
NPAD = 10240
SL = NPAD // NS
CHUNK = 128
G = 8
GS = 8
EP = 327680
RW = EP // (NW * CHUNK)

_mesh = plsc.VectorSubcoreMesh(core_axis_name="c", subcore_axis_name="s")


@functools.partial(
    pl.kernel,
    mesh=_mesh,
    out_type=jax.ShapeDtypeStruct((NC, NPAD), jnp.float32),
    scratch_types=[
        pltpu.VMEM((G, CHUNK), jnp.int32),
        pltpu.VMEM((CHUNK,), jnp.float32),
        pltpu.VMEM_SHARED((NPAD,), jnp.float32),
    ],
)
def _deg_kernel(col_hbm, zeros_hbm, out_hbm, idx_v, ones_v, deg_sh):
    c = lax.axis_index("c")
    s = lax.axis_index("s")
    w = c * NS + s
    for k in range(CHUNK // 16):
        ones_v[pl.ds(k * 16, 16)] = jnp.ones((16,), jnp.float32)
    pltpu.sync_copy(zeros_hbm.at[pl.ds(s * SL, SL)], deg_sh.at[pl.ds(s * SL, SL)])
    plsc.subcore_barrier()

    def outer(i, carry):
        base = w * RW + i * G
        pltpu.sync_copy(col_hbm.at[pl.ds(base, G)], idx_v)
        for j in range(G):
            pltpu.sync_copy(ones_v, deg_sh.at[idx_v.at[j]], add=True)
        return carry

    lax.fori_loop(0, RW // G, outer, 0)
    plsc.subcore_barrier()
    pltpu.sync_copy(deg_sh.at[pl.ds(s * SL, SL)], out_hbm.at[c, pl.ds(s * SL, SL)])


@functools.partial(
    pl.kernel,
    mesh=_mesh,
    out_type=(
        jax.ShapeDtypeStruct((NC, NPAD, H), jnp.float32),
        jax.ShapeDtypeStruct((NC, NPAD), jnp.float32),
    ),
    scratch_types=[
        pltpu.VMEM((GS, CHUNK), jnp.int32),
        pltpu.VMEM((GS, CHUNK), jnp.int32),
        pltpu.VMEM((GS * CHUNK, H), jnp.float32),
        pltpu.VMEM((GS * CHUNK,), jnp.float32),
        pltpu.VMEM_SHARED((NPAD, H), jnp.float32),
        pltpu.VMEM_SHARED((NPAD,), jnp.float32),
        pltpu.SemaphoreType.DMA,
        pltpu.SemaphoreType.DMA,
    ],
    compiler_params=pltpu.CompilerParams(use_tc_tiling_on_sc=False),
)
def _scatter_kernel(row_hbm, col_hbm, y_hbm, dinv_hbm, z1_hbm, z2_hbm,
                    acc_out, t_out,
                    ridx_v, cidx_v, rows_v, dv_v, acc_sh, t_sh, sem, sem2):
    c = lax.axis_index("c")
    s = lax.axis_index("s")
    w = c * NS + s
    pltpu.sync_copy(z2_hbm.at[pl.ds(s * SL, SL)], acc_sh.at[pl.ds(s * SL, SL)])
    pltpu.sync_copy(z1_hbm.at[pl.ds(s * SL, SL)], t_sh.at[pl.ds(s * SL, SL)])
    plsc.subcore_barrier()

    def outer(i, carry):
        base = w * RW + i * GS
        pltpu.sync_copy(row_hbm.at[pl.ds(base, GS)], ridx_v)
        pltpu.sync_copy(col_hbm.at[pl.ds(base, GS)], cidx_v)
        cps = []
        for j in range(GS):
            cps.append(pltpu.async_copy(
                y_hbm.at[ridx_v.at[j]],
                rows_v.at[pl.ds(j * CHUNK, CHUNK)], sem))
            cps.append(pltpu.async_copy(
                dinv_hbm.at[cidx_v.at[j]],
                dv_v.at[pl.ds(j * CHUNK, CHUNK)], sem2))
        for cp in cps:
            cp.wait()
        for j in range(GS):
            pltpu.sync_copy(rows_v.at[pl.ds(j * CHUNK, CHUNK)],
                            acc_sh.at[cidx_v.at[j]], add=True)
            pltpu.sync_copy(dv_v.at[pl.ds(j * CHUNK, CHUNK)],
                            t_sh.at[ridx_v.at[j]], add=True)
        return carry

    lax.fori_loop(0, RW // GS, outer, 0)
    plsc.subcore_barrier()
    pltpu.sync_copy(acc_sh.at[pl.ds(s * SL, SL)], acc_out.at[c, pl.ds(s * SL, SL)])
    pltpu.sync_copy(t_sh.at[pl.ds(s * SL, SL)], t_out.at[c, pl.ds(s * SL, SL)])


def _scale_body(x_ref, w1_ref, degp_ref, y_ref, dinv_ref):
    deg = degp_ref[:, 0:1] + degp_ref[:, 1:2] + 1.0
    dinv = lax.rsqrt(deg)
    dinv_ref[...] = dinv
    xw = jnp.dot(x_ref[...], w1_ref[...], preferred_element_type=jnp.float32)
    y_ref[...] = xw * dinv


_scale_call = pl.pallas_call(
    _scale_body,
    out_shape=(
        jax.ShapeDtypeStruct((NPAD, H), jnp.float32),
        jax.ShapeDtypeStruct((NPAD, 1), jnp.float32),
    ),
)


def _final_body(accp_ref, tp_ref, y_ref, dinv_ref, b1_ref, w2_ref, b2_ref,
                lw_ref, lb_ref, out_ref):
    dinv = dinv_ref[...]
    acc = accp_ref[0] + accp_ref[1]
    h = jnp.maximum(dinv * (acc + y_ref[...]) + b1_ref[...], 0.0)
    t = tp_ref[:, 0:1] + tp_ref[:, 1:2]
    s = dinv * (dinv + t)
    ridx = lax.broadcasted_iota(jnp.int32, (NPAD, 1), 0)
    s = jnp.where(ridx < N, s, 0.0)
    z = jnp.sum(s * h, axis=0, keepdims=True)
    pooled = jnp.dot(z * (1.0 / N), w2_ref[...],
                     preferred_element_type=jnp.float32) + b2_ref[...]
    out_ref[...] = jnp.dot(pooled, lw_ref[...],
                           preferred_element_type=jnp.float32) + lb_ref[...]


_final_call = pl.pallas_call(
    _final_body,
    out_shape=jax.ShapeDtypeStruct((1, OUT), jnp.float32),
)


def kernel(x, edge_index, W1, b1, W2, b2, lin_W, lin_b):
    row = edge_index[0]
    col = edge_index[1]
    npad_extra = NPAD - N
    pad_idx = (N + jnp.arange(EP - E, dtype=jnp.int32) % npad_extra)
    rowp = jnp.concatenate([row, pad_idx]).reshape(EP // CHUNK, CHUNK)
    colp = jnp.concatenate([col, pad_idx]).reshape(EP // CHUNK, CHUNK)
    zeros1 = jnp.zeros((NPAD,), jnp.float32)
    zeros2 = jnp.zeros((NPAD, H), jnp.float32)

    degp = _deg_kernel(colp, zeros1)
    xp = jnp.pad(x, ((0, npad_extra), (0, 0)))
    y, dinv2 = _scale_call(xp, W1, degp.T)
    dinv1 = dinv2.reshape(NPAD)
    accp, tp = _scatter_kernel(rowp, colp, y, dinv1, zeros1, zeros2)
    out = _final_call(accp, tp.T, y, dinv2, b1.reshape(1, H), W2,
                      b2.reshape(1, H), lin_W, lin_b.reshape(1, OUT))
    return out.reshape(OUT)
